# baseline (speedup 1.0000x reference)
import functools

import jax
import jax.numpy as jnp
from jax import lax
from jax.experimental import pallas as pl
from jax.experimental.pallas import tpu as pltpu
from jax.experimental.pallas import tpu_sc as plsc

NC, NS = 2, 16
NW = NC * NS
GW = 128
GPS = 8
STEP_ROWS = GW * GPS


@functools.partial(jax.jit, static_argnames=("B", "D"))
def _sc_gather(table, idx, B, D):
    b_per_w = B // NW
    n_steps = b_per_w // STEP_ROWS
    idx3 = idx.reshape(NW, b_per_w // GW, GW)
    mesh = plsc.VectorSubcoreMesh(core_axis_name="c", subcore_axis_name="s")

    @functools.partial(
        pl.kernel,
        out_type=jax.ShapeDtypeStruct((B, D), jnp.float32),
        mesh=mesh,
        scratch_types=[
            pltpu.VMEM((b_per_w // GW, GW), jnp.int32),
            pltpu.VMEM((STEP_ROWS, D), jnp.float32),
            pltpu.SemaphoreType.DMA,
        ],
        compiler_params=pltpu.CompilerParams(use_tc_tiling_on_sc=False),
    )
    def k(table_hbm, idx_hbm, out_hbm, idx_v, rows_v, sem):
        wid = lax.axis_index("s") * NC + lax.axis_index("c")
        pltpu.sync_copy(idx_hbm.at[wid], idx_v)
        base = wid * b_per_w

        @pl.loop(0, n_steps)
        def step(g):
            cps = [
                pltpu.async_copy(
                    table_hbm.at[idx_v.at[g * GPS + j]],
                    rows_v.at[pl.ds(j * GW, GW)],
                    sem,
                )
                for j in range(GPS)
            ]
            for cp in cps:
                cp.wait()
            pltpu.sync_copy(
                rows_v, out_hbm.at[pl.ds(base + g * STEP_ROWS, STEP_ROWS)]
            )

    return k(table, idx3)


def kernel(x, table):
    B = x.size
    D = table.shape[1]
    idx = x.reshape(-1).astype(jnp.int32)
    out = _sc_gather(table, idx, B, D)
    return out.reshape(x.shape + (D,))


# speedup vs baseline: 1.0089x; 1.0089x over previous
import functools

import jax
import jax.numpy as jnp
from jax import lax
from jax.experimental import pallas as pl
from jax.experimental.pallas import tpu as pltpu
from jax.experimental.pallas import tpu_sc as plsc

NC, NS = 2, 16
NW = NC * NS
GW = 128
GPS = 5
STEP = GW * GPS


@functools.partial(jax.jit, static_argnames=("B", "D"))
def _sc_gather(table, idx, B, D):
    b_per_w = B // NW
    n = b_per_w // STEP
    assert n >= 4 and n % 2 == 0
    idx3 = idx.reshape(NW, b_per_w // GW, GW)
    mesh = plsc.VectorSubcoreMesh(core_axis_name="c", subcore_axis_name="s")

    @functools.partial(
        pl.kernel,
        out_type=jax.ShapeDtypeStruct((B, D), jnp.float32),
        mesh=mesh,
        scratch_types=[
            pltpu.VMEM((b_per_w // GW, GW), jnp.int32),
            pltpu.VMEM((2 * STEP, D), jnp.float32),
            pltpu.SemaphoreType.DMA,
            pltpu.SemaphoreType.DMA,
            pltpu.SemaphoreType.DMA,
            pltpu.SemaphoreType.DMA,
        ],
        compiler_params=pltpu.CompilerParams(use_tc_tiling_on_sc=False),
    )
    def k(table_hbm, idx_hbm, out_hbm, idx_v, rows_v, g0, g1, w0, w1):
        wid = lax.axis_index("s") * NC + lax.axis_index("c")
        pltpu.sync_copy(idx_hbm.at[wid], idx_v)
        base = wid * b_per_w
        gsem = (g0, g1)
        wsem = (w0, w1)

        def fire_g(g, par):
            off = par * STEP
            for j in range(GPS):
                pltpu.async_copy(
                    table_hbm.at[idx_v.at[g * GPS + j]],
                    rows_v.at[pl.ds(off + j * GW, GW)],
                    gsem[par],
                )

        def drain_g(par):
            pltpu.make_async_copy(
                table_hbm.at[pl.ds(0, STEP)],
                rows_v.at[pl.ds(par * STEP, STEP)],
                gsem[par],
            ).wait()

        def fire_w(g, par):
            pltpu.async_copy(
                rows_v.at[pl.ds(par * STEP, STEP)],
                out_hbm.at[pl.ds(base + g * STEP, STEP)],
                wsem[par],
            )

        def drain_w(par):
            pltpu.make_async_copy(
                rows_v.at[pl.ds(par * STEP, STEP)],
                out_hbm.at[pl.ds(base, STEP)],
                wsem[par],
            ).wait()

        fire_g(0, 0)
        fire_g(1, 1)
        drain_g(0)
        fire_w(0, 0)

        @pl.loop(0, (n - 4) // 2)
        def superstep(s):
            godd = 2 * s + 1
            drain_w(0)
            fire_g(godd + 1, 0)
            drain_g(1)
            fire_w(godd, 1)
            drain_w(1)
            fire_g(godd + 2, 1)
            drain_g(0)
            fire_w(godd + 1, 0)

        drain_w(0)
        fire_g(n - 2, 0)
        drain_g(1)
        fire_w(n - 3, 1)
        drain_w(1)
        fire_g(n - 1, 1)
        drain_g(0)
        fire_w(n - 2, 0)
        drain_g(1)
        fire_w(n - 1, 1)
        drain_w(0)
        drain_w(1)

    return k(table, idx3)


def kernel(x, table):
    B = x.size
    D = table.shape[1]
    idx = x.reshape(-1).astype(jnp.int32)
    out = _sc_gather(table, idx, B, D)
    return out.reshape(x.shape + (D,))


# speedup vs baseline: 1.0094x; 1.0005x over previous
import functools

import jax
import jax.numpy as jnp
from jax import lax
from jax.experimental import pallas as pl
from jax.experimental.pallas import tpu as pltpu
from jax.experimental.pallas import tpu_sc as plsc

NC, NS = 2, 16
NW = NC * NS
GW = 100
GPS = 4
CB0 = 2


@functools.partial(jax.jit, static_argnames=("B0", "B1", "D"))
def _sc_gather(table, idx, B0, B1, D):
    b0_per_w = B0 // NW
    b_per_w = b0_per_w * B1
    step = CB0 * B1
    n = b0_per_w // CB0
    assert n >= 4 and n % 2 == 0 and step == GW * GPS
    idx3 = idx.reshape(NW, b_per_w // GW, GW)
    mesh = plsc.VectorSubcoreMesh(core_axis_name="c", subcore_axis_name="s")

    @functools.partial(
        pl.kernel,
        out_type=jax.ShapeDtypeStruct((B0, B1, D), jnp.float32),
        mesh=mesh,
        scratch_types=[
            pltpu.VMEM((b_per_w // GW, GW), jnp.int32),
            pltpu.VMEM((2, CB0, B1, D), jnp.float32),
            pltpu.SemaphoreType.DMA,
            pltpu.SemaphoreType.DMA,
            pltpu.SemaphoreType.DMA,
            pltpu.SemaphoreType.DMA,
        ],
        compiler_params=pltpu.CompilerParams(use_tc_tiling_on_sc=False),
    )
    def k(table_hbm, idx_hbm, out_hbm, idx_v, rows_v, g0, g1, w0, w1):
        wid = lax.axis_index("s") * NC + lax.axis_index("c")
        pltpu.sync_copy(idx_hbm.at[wid], idx_v)
        b0_base = wid * b0_per_w
        gsem = (g0, g1)
        wsem = (w0, w1)

        def fire_g(g, par):
            for j in range(GPS):
                pltpu.async_copy(
                    table_hbm.at[idx_v.at[g * GPS + j]],
                    rows_v.at[par, (j * GW) // B1, pl.ds((j * GW) % B1, GW)],
                    gsem[par],
                )

        def drain_g(par):
            pltpu.make_async_copy(
                out_hbm.at[pl.ds(0, CB0)], rows_v.at[par], gsem[par]
            ).wait()

        def fire_w(g, par):
            pltpu.async_copy(
                rows_v.at[par],
                out_hbm.at[pl.ds(b0_base + g * CB0, CB0)],
                wsem[par],
            )

        def drain_w(par):
            pltpu.make_async_copy(
                rows_v.at[par], out_hbm.at[pl.ds(0, CB0)], wsem[par]
            ).wait()

        fire_g(0, 0)
        fire_g(1, 1)
        drain_g(0)
        fire_w(0, 0)

        @pl.loop(0, (n - 4) // 2)
        def superstep(s):
            godd = 2 * s + 1
            drain_w(0)
            fire_g(godd + 1, 0)
            drain_g(1)
            fire_w(godd, 1)
            drain_w(1)
            fire_g(godd + 2, 1)
            drain_g(0)
            fire_w(godd + 1, 0)

        drain_w(0)
        fire_g(n - 2, 0)
        drain_g(1)
        fire_w(n - 3, 1)
        drain_w(1)
        fire_g(n - 1, 1)
        drain_g(0)
        fire_w(n - 2, 0)
        drain_g(1)
        fire_w(n - 1, 1)
        drain_w(0)
        drain_w(1)

    return k(table, idx3)


def kernel(x, table):
    B0, B1 = x.shape
    D = table.shape[1]
    idx = x.reshape(-1).astype(jnp.int32)
    return _sc_gather(table, idx, B0, B1, D)


# speedup vs baseline: 1.2310x; 1.2195x over previous
import functools

import jax
import jax.numpy as jnp
from jax import lax
from jax.experimental import pallas as pl
from jax.experimental.pallas import tpu as pltpu
from jax.experimental.pallas import tpu_sc as plsc

NC, NS = 2, 16
NW = NC * NS
GW = 100
GPS = 2


@functools.partial(jax.jit, static_argnames=("B0", "B1", "D"))
def _sc_gather(tablep, idx, B0, B1, D):
    b0_per_w = B0 // NW
    b_per_w = b0_per_w * B1
    n = b0_per_w
    assert B1 == GW * GPS and n >= 4 and n % 2 == 0
    idx3 = idx.reshape(NW, b_per_w // GW, GW)
    mesh = plsc.VectorSubcoreMesh(core_axis_name="c", subcore_axis_name="s")

    @functools.partial(
        pl.kernel,
        out_type=jax.ShapeDtypeStruct((B0, B1, 2 * D), jnp.float32),
        mesh=mesh,
        scratch_types=[
            pltpu.VMEM((b_per_w // GW, GW), jnp.int32),
            pltpu.VMEM((2, B1, 2 * D), jnp.float32),
            pltpu.SemaphoreType.DMA,
            pltpu.SemaphoreType.DMA,
            pltpu.SemaphoreType.DMA,
            pltpu.SemaphoreType.DMA,
        ],
        compiler_params=pltpu.CompilerParams(use_tc_tiling_on_sc=True),
    )
    def k(table_hbm, idx_hbm, out_hbm, idx_v, rows_v, g0, g1, w0, w1):
        wid = lax.axis_index("s") * NC + lax.axis_index("c")
        pltpu.sync_copy(idx_hbm.at[wid], idx_v)
        b0_base = wid * b0_per_w
        gsem = (g0, g1)
        wsem = (w0, w1)

        def fire_g(g, par):
            for j in range(GPS):
                pltpu.async_copy(
                    table_hbm.at[idx_v.at[g * GPS + j]],
                    rows_v.at[par, pl.ds(j * GW, GW)],
                    gsem[par],
                )

        def drain_g(par):
            pltpu.make_async_copy(
                table_hbm.at[pl.ds(0, B1)], rows_v.at[par], gsem[par]
            ).wait()

        def fire_w(g, par):
            pltpu.async_copy(
                rows_v.at[par],
                out_hbm.at[b0_base + g],
                wsem[par],
            )

        def drain_w(par):
            pltpu.make_async_copy(
                rows_v.at[par], out_hbm.at[0], wsem[par]
            ).wait()

        fire_g(0, 0)
        fire_g(1, 1)
        drain_g(0)
        fire_w(0, 0)

        @pl.loop(0, (n - 4) // 2)
        def superstep(s):
            godd = 2 * s + 1
            drain_w(0)
            fire_g(godd + 1, 0)
            drain_g(1)
            fire_w(godd, 1)
            drain_w(1)
            fire_g(godd + 2, 1)
            drain_g(0)
            fire_w(godd + 1, 0)

        drain_w(0)
        fire_g(n - 2, 0)
        drain_g(1)
        fire_w(n - 3, 1)
        drain_w(1)
        fire_g(n - 1, 1)
        drain_g(0)
        fire_w(n - 2, 0)
        drain_g(1)
        fire_w(n - 1, 1)
        drain_w(0)
        drain_w(1)

    return k(tablep, idx3)


def kernel(x, table):
    B0, B1 = x.shape
    V, D = table.shape
    xf = x.reshape(-1).astype(jnp.int32)
    tablep = jnp.pad(table, ((0, 0), (0, D)))
    out2 = _sc_gather(tablep, xf, B0, B1, D)
    return out2[:, :, :D]
